# baseline (speedup 1.0000x reference)
import jax
import jax.numpy as jnp
from jax.experimental import pallas as pl


_TI = 128
_BB = 2


def _edge_kernel(x_ref, m_ref, w1_ref, b1_ref, w2_ref, b2_ref,
                 w3_ref, b3_ref, out_ref):
    n = x_ref.shape[1]
    f = x_ref.shape[2]
    w1a = w1_ref[:, :f]
    w1b = w1_ref[:, f:]
    b1 = b1_ref[...].T
    w2 = w2_ref[...].astype(jnp.bfloat16)
    b2 = b2_ref[...].T
    w3 = w3_ref[...]
    b3 = b3_ref[0, 0]
    ri = jax.lax.broadcasted_iota(jnp.int32, (n, n), 0)
    ci = jax.lax.broadcasted_iota(jnp.int32, (n, n), 1)

    contract_f = (((1,), (1,)), ((), ()))

    for b in range(_BB):
        x = x_ref[b]
        at = jax.lax.dot_general(
            w1a, x, contract_f, preferred_element_type=jnp.float32) + b1
        bt = jax.lax.dot_general(
            w1b, x, contract_f, preferred_element_type=jnp.float32)
        at16 = at.astype(jnp.bfloat16)
        bt16 = bt.astype(jnp.bfloat16)

        rows = []
        for i0 in range(0, n, _TI):
            chunks = [jnp.maximum(at16[:, ii:ii + 1] + bt16, 0)
                      for ii in range(i0, i0 + _TI)]
            h1 = jnp.concatenate(chunks, axis=1)
            h2 = jnp.maximum(
                jnp.dot(w2, h1, preferred_element_type=jnp.float32) + b2, 0.0)
            z = jnp.dot(w3, h2, preferred_element_type=jnp.float32)
            rows.append(z.astype(jnp.bfloat16).reshape(_TI, n))
        zmat = jnp.concatenate(rows, axis=0).astype(jnp.float32)
        score = jax.nn.sigmoid(zmat + b3)

        m = m_ref[b, 0].astype(jnp.float32)
        pair = m[:, None] * m[None, :]
        adj = jnp.where(ri == ci, 0.0, score * pair)
        out_ref[b] = (adj + adj.T) * 0.5


@jax.jit
def kernel(node_features, node_masks, W1, b1, W2, b2, W3, b3):
    B, N, F = node_features.shape
    masks = node_masks.reshape(B, 1, N)

    full = lambda shape: pl.BlockSpec(shape, lambda i: (0,) * len(shape))
    out = pl.pallas_call(
        _edge_kernel,
        grid=(B // _BB,),
        in_specs=[
            pl.BlockSpec((_BB, N, F), lambda i: (i, 0, 0)),
            pl.BlockSpec((_BB, 1, N), lambda i: (i, 0, 0)),
            full((64, 2 * F)),
            full((1, 64)),
            full((32, 64)),
            full((1, 32)),
            full((1, 32)),
            full((1, 1)),
        ],
        out_specs=pl.BlockSpec((_BB, N, N), lambda i: (i, 0, 0)),
        out_shape=jax.ShapeDtypeStruct((B, N, N), jnp.float32),
    )(node_features, masks, W1, b1.reshape(1, 64), W2, b2.reshape(1, 32),
      W3, b3.reshape(1, 1))
    return out


# speedup vs baseline: 1.0619x; 1.0619x over previous
import jax
import jax.numpy as jnp
from jax.experimental import pallas as pl


_TI = 64
_BB = 4


def _edge_kernel(x_ref, m_ref, w1_ref, b1_ref, w2_ref, b2_ref,
                 w3_ref, b3_ref, out_ref):
    n = x_ref.shape[1]
    f = x_ref.shape[2]
    w1a = w1_ref[:, :f]
    w1b = w1_ref[:, f:]
    b1 = b1_ref[...].T
    w2 = w2_ref[...].astype(jnp.bfloat16)
    b2 = b2_ref[...].T
    w3 = w3_ref[...]
    b3 = b3_ref[0, 0]
    ri = jax.lax.broadcasted_iota(jnp.int32, (n, n), 0)
    ci = jax.lax.broadcasted_iota(jnp.int32, (n, n), 1)

    contract_f = (((1,), (1,)), ((), ()))

    for b in range(_BB):
        x = x_ref[b]
        at = jax.lax.dot_general(
            w1a, x, contract_f, preferred_element_type=jnp.float32) + b1
        bt = jax.lax.dot_general(
            w1b, x, contract_f, preferred_element_type=jnp.float32)
        at16 = at.astype(jnp.bfloat16)
        bt16 = bt.astype(jnp.bfloat16)

        rows = []
        for i0 in range(0, n, _TI):
            chunks = [jnp.maximum(at16[:, ii:ii + 1] + bt16, 0)
                      for ii in range(i0, i0 + _TI)]
            h1 = jnp.concatenate(chunks, axis=1)
            h2 = jnp.maximum(
                jnp.dot(w2, h1, preferred_element_type=jnp.float32) + b2, 0.0)
            z = jnp.dot(w3, h2, preferred_element_type=jnp.float32)
            rows.append(z.astype(jnp.bfloat16).reshape(_TI, n))
        zmat = jnp.concatenate(rows, axis=0).astype(jnp.float32)
        score = jax.nn.sigmoid(zmat + b3)

        m = m_ref[b, 0].astype(jnp.float32)
        pair = m[:, None] * m[None, :]
        adj = jnp.where(ri == ci, 0.0, score * pair)
        out_ref[b] = (adj + adj.T) * 0.5


@jax.jit
def kernel(node_features, node_masks, W1, b1, W2, b2, W3, b3):
    B, N, F = node_features.shape
    masks = node_masks.reshape(B, 1, N)

    full = lambda shape: pl.BlockSpec(shape, lambda i: (0,) * len(shape))
    out = pl.pallas_call(
        _edge_kernel,
        grid=(B // _BB,),
        in_specs=[
            pl.BlockSpec((_BB, N, F), lambda i: (i, 0, 0)),
            pl.BlockSpec((_BB, 1, N), lambda i: (i, 0, 0)),
            full((64, 2 * F)),
            full((1, 64)),
            full((32, 64)),
            full((1, 32)),
            full((1, 32)),
            full((1, 1)),
        ],
        out_specs=pl.BlockSpec((_BB, N, N), lambda i: (i, 0, 0)),
        out_shape=jax.ShapeDtypeStruct((B, N, N), jnp.float32),
    )(node_features, masks, W1, b1.reshape(1, 64), W2, b2.reshape(1, 32),
      W3, b3.reshape(1, 1))
    return out
